# baseline (speedup 1.0000x reference)
import functools

import jax
import jax.numpy as jnp
from jax import lax
from jax.experimental import pallas as pl
from jax.experimental.pallas import tpu as pltpu
from jax.experimental.pallas import tpu_sc as plsc

_NC = 2
_NS = 16
_LANES = 16


def _sc_pair_accum(attr_pad, pid_pad, w1v, b1v, *, ppad, epad, rpt):
    mesh = plsc.VectorSubcoreMesh(core_axis_name="c", subcore_axis_name="s")

    @functools.partial(
        pl.kernel,
        mesh=mesh,
        out_type=jax.ShapeDtypeStruct((ppad, 2 * _LANES), jnp.float32),
        scratch_types=[
            pltpu.VMEM((epad,), jnp.int32),
            pltpu.VMEM((epad,), jnp.float32),
            pltpu.VMEM((rpt + 8, 2 * _LANES), jnp.float32),
            pltpu.VMEM((_LANES,), jnp.float32),
            pltpu.VMEM((_LANES,), jnp.float32),
        ],
    )
    def k(attr_hbm, pid_hbm, w1_hbm, b1_hbm, out_hbm,
          pid_v, attr_v, acc_v, w1_v, b1_v):
        c = lax.axis_index("c")
        s = lax.axis_index("s")
        wid = c * _NS + s
        rbase = wid * rpt
        pltpu.sync_copy(attr_hbm, attr_v)
        pltpu.sync_copy(pid_hbm, pid_v)
        pltpu.sync_copy(w1_hbm, w1_v)
        pltpu.sync_copy(b1_hbm, b1_v)
        w1r = w1_v[...]
        b1r = b1_v[...]
        cntv = jnp.where(lax.iota(jnp.int32, _LANES) == 0, 1.0, 0.0)
        zv = jnp.zeros((_LANES,), jnp.float32)

        def zbody(r, carry):
            acc_v[r, 0:_LANES] = zv
            acc_v[r, _LANES:2 * _LANES] = zv
            return carry
        lax.fori_loop(0, rpt + 8, zbody, 0)

        def ebody(g, carry):
            av = attr_v[pl.ds(g * _LANES, _LANES)]
            pv = pid_v[pl.ds(g * _LANES, _LANES)]
            lv = pv - rbase
            inr = (lv >= 0) & (lv < rpt)
            iv = jnp.where(inr, lv, rpt)
            for j in range(_LANES):
                a = jnp.full((_LANES,), av[j], jnp.float32)
                h = jnp.maximum(a * w1r + b1r, 0.0)
                idx = iv[j]
                plsc.addupdate(acc_v.at[idx, 0:_LANES], h)
                plsc.addupdate(acc_v.at[idx, _LANES:2 * _LANES], cntv)
            return carry
        lax.fori_loop(0, epad // _LANES, ebody, 0)

        pltpu.sync_copy(acc_v.at[pl.ds(0, rpt)],
                        out_hbm.at[pl.ds(rbase, rpt)])

    return k(attr_pad, pid_pad, w1v, b1v)


def _tc_build_w(hraw, w2r, b2r, *, in_ch, out_ch, ppad, npair):
    ic = 4

    def body(h_ref, w2_ref, b2_ref, o_ref):
        h = h_ref[...]
        h16 = h[:, :_LANES]
        cnt = h[:, _LANES:_LANES + 1]
        for j in range(ic):
            wi = jnp.dot(h16, w2_ref[j], preferred_element_type=jnp.float32)
            o_ref[j] = (wi + cnt * b2_ref[j])[:o_ref.shape[1]].astype(
                jnp.bfloat16)

    return pl.pallas_call(
        body,
        grid=(in_ch // ic,),
        in_specs=[
            pl.BlockSpec((ppad, 2 * _LANES), lambda i: (0, 0)),
            pl.BlockSpec((ic, _LANES, out_ch), lambda i: (i, 0, 0)),
            pl.BlockSpec((ic, 1, out_ch), lambda i: (i, 0, 0)),
        ],
        out_specs=pl.BlockSpec((ic, npair, out_ch), lambda i: (i, 0, 0)),
        out_shape=jax.ShapeDtypeStruct((in_ch, npair, out_ch), jnp.bfloat16),
    )(hraw, w2r, b2r)


def _tc_matmul(st2, wbig, *, bb):
    b, k = st2.shape
    _, nxo = wbig.shape

    def body(x_ref, w_ref, o_ref):
        o_ref[...] = jnp.dot(x_ref[...], w_ref[...],
                             preferred_element_type=jnp.float32)

    return pl.pallas_call(
        body,
        grid=(b // bb,),
        in_specs=[
            pl.BlockSpec((bb, k), lambda i: (i, 0)),
            pl.BlockSpec((k, nxo), lambda i: (0, 0)),
        ],
        out_specs=pl.BlockSpec((bb, nxo), lambda i: (i, 0)),
        out_shape=jax.ShapeDtypeStruct((b, nxo), jnp.float32),
    )(st2, wbig)


def _tc_epilogue(state, pos_feat, agg3, root_w, root_b, lw1, lb1, lw2, lb2,
                 lw3, lb3, *, bb):
    b, n, c_raw = state.shape
    p = pos_feat.shape[-1]
    in_ch = c_raw + p
    out_ch = agg3.shape[-1]
    hid = lw2.shape[0]
    nb = b // bb

    def body(s_ref, pos_ref, a_ref, rw_ref, rb_ref, lw1_ref, lb1_ref,
             lw2_ref, lb2_ref, lw3_ref, lb3_ref, act_ref, part_ref):
        aggf = a_ref[...].reshape(bb * n, out_ch)
        posb = jnp.broadcast_to(pos_ref[...][None], (bb, n, p))
        s3 = jnp.concatenate([s_ref[...], posb], axis=-1)
        sf = s3.reshape(bb * n, in_ch)
        root = jnp.dot(sf, rw_ref[...], preferred_element_type=jnp.float32)
        out1 = jnp.maximum(root + rb_ref[...] + aggf, 0.0)

        t = s3[:, :, 1:2]
        total = jnp.sum(t, axis=1, keepdims=True)
        totf = jnp.broadcast_to(total, (bb, n, 1)).reshape(bb * n, 1)

        lw1v = lw1_ref[...]
        x1 = (jnp.dot(out1, lw1v[:out_ch], preferred_element_type=jnp.float32)
              + totf * lw1v[out_ch:out_ch + 1]
              + jnp.dot(sf, lw1v[out_ch + 1:], preferred_element_type=jnp.float32)
              + lb1_ref[...])
        x1 = jnp.where(x1 > 0, x1, 0.01 * x1)
        x2 = jnp.dot(x1, lw2_ref[...], preferred_element_type=jnp.float32) + lb2_ref[...]
        x2 = jnp.where(x2 > 0, x2, 0.01 * x2)
        x3 = jnp.dot(x2, lw3_ref[...], preferred_element_type=jnp.float32) + lb3_ref[...]
        conc = jnp.maximum(x3, 0.0) + jnp.log(1.0 + jnp.exp(-jnp.abs(x3)))
        conc3 = conc.reshape(bb, n, 1)
        denom = jnp.sum(conc3, axis=1, keepdims=True) + 1e-20
        act_ref[...] = conc3 / denom
        part_ref[...] = jnp.full((1, 1, 128), jnp.sum(jnp.abs(conc3)),
                                 jnp.float32)

    return pl.pallas_call(
        body,
        grid=(nb,),
        in_specs=[
            pl.BlockSpec((bb, n, c_raw), lambda i: (i, 0, 0)),
            pl.BlockSpec((n, p), lambda i: (0, 0)),
            pl.BlockSpec((bb, n, out_ch), lambda i: (i, 0, 0)),
            pl.BlockSpec((in_ch, out_ch), lambda i: (0, 0)),
            pl.BlockSpec((1, out_ch), lambda i: (0, 0)),
            pl.BlockSpec((in_ch + out_ch + 1, hid), lambda i: (0, 0)),
            pl.BlockSpec((1, hid), lambda i: (0, 0)),
            pl.BlockSpec((hid, hid), lambda i: (0, 0)),
            pl.BlockSpec((1, hid), lambda i: (0, 0)),
            pl.BlockSpec((hid, 1), lambda i: (0, 0)),
            pl.BlockSpec((1, 1), lambda i: (0, 0)),
        ],
        out_specs=[
            pl.BlockSpec((bb, n, 1), lambda i: (i, 0, 0)),
            pl.BlockSpec((1, 1, 128), lambda i: (i, 0, 0)),
        ],
        out_shape=[
            jax.ShapeDtypeStruct((b, n, 1), jnp.float32),
            jax.ShapeDtypeStruct((nb, 1, 128), jnp.float32),
        ],
    )(state, pos_feat, agg3, root_w, root_b, lw1, lb1, lw2, lb2, lw3, lb3)


def kernel(state, edge_index, edge_attr, pos_feat, w1, b1, w2, b2,
           root_w, root_b, lw1, lb1, lw2, lb2, lw3, lb3):
    b, n, c_raw = state.shape
    p = pos_feat.shape[-1]
    in_ch = c_raw + p
    out_ch = root_w.shape[1]
    e = edge_index.shape[1]
    hid = lw2.shape[0]
    npair = n * n
    nw = _NC * _NS

    epad = -(-e // _LANES) * _LANES
    rpt = -(-npair // (nw * 8)) * 8
    ppad = nw * rpt

    src = edge_index[0].astype(jnp.int32)
    dst = edge_index[1].astype(jnp.int32)
    pid = src * n + dst
    pid_pad = jnp.concatenate(
        [pid, jnp.full((epad - e,), npair, jnp.int32)])
    attr_pad = jnp.concatenate(
        [edge_attr[:, 0], jnp.zeros((epad - e,), jnp.float32)])

    hraw = _sc_pair_accum(attr_pad, pid_pad, w1.reshape(_LANES),
                          b1.reshape(_LANES), ppad=ppad, epad=epad, rpt=rpt)

    w2r = w2.reshape(_LANES, in_ch, out_ch).transpose(1, 0, 2)
    b2r = b2.reshape(in_ch, 1, out_ch)
    wflat = _tc_build_w(hraw, w2r, b2r, in_ch=in_ch, out_ch=out_ch,
                        ppad=ppad, npair=npair)
    wbig = wflat.reshape(in_ch * n, n * out_ch)

    pos_b = jnp.broadcast_to(pos_feat[None], (b, n, p))
    s = jnp.concatenate([state, pos_b], axis=-1)
    st2 = s.transpose(0, 2, 1).reshape(b, in_ch * n).astype(jnp.bfloat16)
    agg = _tc_matmul(st2, wbig, bb=128)
    agg3 = agg.reshape(b, n, out_ch)

    act3, parts = _tc_epilogue(
        state, pos_feat, agg3, root_w, root_b.reshape(1, out_ch), lw1,
        lb1.reshape(1, hid), lw2, lb2.reshape(1, hid), lw3,
        lb3.reshape(1, 1), bb=128)
    action = act3.reshape(b, n)
    regularize = jnp.sum(parts[:, 0, 0]) / (b * n)
    return (action, regularize)


# speedup vs baseline: 1.0724x; 1.0724x over previous
import functools

import jax
import jax.numpy as jnp
from jax import lax
from jax.experimental import pallas as pl
from jax.experimental.pallas import tpu as pltpu
from jax.experimental.pallas import tpu_sc as plsc

_NC = 2
_NS = 16
_LANES = 16


def _sc_pair_accum(attr_pad, pid_pad, w1v, b1v, *, ppad, epad, rpt):
    mesh = plsc.VectorSubcoreMesh(core_axis_name="c", subcore_axis_name="s")

    @functools.partial(
        pl.kernel,
        mesh=mesh,
        out_type=jax.ShapeDtypeStruct((ppad, 2 * _LANES), jnp.float32),
        scratch_types=[
            pltpu.VMEM((epad,), jnp.int32),
            pltpu.VMEM((epad,), jnp.float32),
            pltpu.VMEM((rpt + 8, 2 * _LANES), jnp.float32),
            pltpu.VMEM((_LANES,), jnp.float32),
            pltpu.VMEM((_LANES,), jnp.float32),
        ],
    )
    def k(attr_hbm, pid_hbm, w1_hbm, b1_hbm, out_hbm,
          pid_v, attr_v, acc_v, w1_v, b1_v):
        c = lax.axis_index("c")
        s = lax.axis_index("s")
        wid = c * _NS + s
        rbase = wid * rpt
        pltpu.sync_copy(attr_hbm, attr_v)
        pltpu.sync_copy(pid_hbm, pid_v)
        pltpu.sync_copy(w1_hbm, w1_v)
        pltpu.sync_copy(b1_hbm, b1_v)
        w1r = w1_v[...]
        b1r = b1_v[...]
        cntv = jnp.where(lax.iota(jnp.int32, _LANES) == 0, 1.0, 0.0)
        zv = jnp.zeros((_LANES,), jnp.float32)

        def zbody(r, carry):
            acc_v[r, 0:_LANES] = zv
            acc_v[r, _LANES:2 * _LANES] = zv
            return carry
        lax.fori_loop(0, rpt + 8, zbody, 0)

        def ebody(g, carry):
            av = attr_v[pl.ds(g * _LANES, _LANES)]
            pv = pid_v[pl.ds(g * _LANES, _LANES)]
            lv = pv - rbase
            inr = (lv >= 0) & (lv < rpt)
            iv = jnp.where(inr, lv, rpt)
            for j in range(_LANES):
                a = jnp.full((_LANES,), av[j], jnp.float32)
                h = jnp.maximum(a * w1r + b1r, 0.0)
                idx = iv[j]
                plsc.addupdate(acc_v.at[idx, 0:_LANES], h)
                plsc.addupdate(acc_v.at[idx, _LANES:2 * _LANES], cntv)
            return carry
        lax.fori_loop(0, epad // _LANES, ebody, 0)

        pltpu.sync_copy(acc_v.at[pl.ds(0, rpt)],
                        out_hbm.at[pl.ds(rbase, rpt)])

    return k(attr_pad, pid_pad, w1v, b1v)


def _tc_build_w(hraw, w2r, b2r, *, in_ch, out_ch, ppad, npair):
    ic = 4

    def body(h_ref, w2_ref, b2_ref, o_ref):
        h = h_ref[...]
        h16 = h[:, :_LANES]
        cnt = h[:, _LANES:_LANES + 1]
        for j in range(ic):
            wi = jnp.dot(h16, w2_ref[j], preferred_element_type=jnp.float32)
            o_ref[j] = (wi + cnt * b2_ref[j])[:o_ref.shape[1]].astype(
                jnp.bfloat16)

    return pl.pallas_call(
        body,
        grid=(in_ch // ic,),
        in_specs=[
            pl.BlockSpec((ppad, 2 * _LANES), lambda i: (0, 0)),
            pl.BlockSpec((ic, _LANES, out_ch), lambda i: (i, 0, 0)),
            pl.BlockSpec((ic, 1, out_ch), lambda i: (i, 0, 0)),
        ],
        out_specs=pl.BlockSpec((ic, npair, out_ch), lambda i: (i, 0, 0)),
        out_shape=jax.ShapeDtypeStruct((in_ch, npair, out_ch), jnp.bfloat16),
    )(hraw, w2r, b2r)


def _tc_matmul(st2, wbig, *, bb):
    b, k = st2.shape
    _, nxo = wbig.shape

    def body(x_ref, w_ref, o_ref):
        o_ref[...] = jnp.dot(x_ref[...], w_ref[...],
                             preferred_element_type=jnp.float32)

    return pl.pallas_call(
        body,
        grid=(b // bb,),
        in_specs=[
            pl.BlockSpec((bb, k), lambda i: (i, 0)),
            pl.BlockSpec((k, nxo), lambda i: (0, 0)),
        ],
        out_specs=pl.BlockSpec((bb, nxo), lambda i: (i, 0)),
        out_shape=jax.ShapeDtypeStruct((b, nxo), jnp.float32),
    )(st2, wbig)


def _tc_epilogue(state, pos_feat, agg3, root_w, root_b, lw1, lb1, lw2, lb2,
                 lw3, lb3, *, bb):
    b, n, c_raw = state.shape
    p = pos_feat.shape[-1]
    in_ch = c_raw + p
    out_ch = agg3.shape[-1]
    hid = lw2.shape[0]
    nb = b // bb

    def body(s_ref, pos_ref, a_ref, rw_ref, rb_ref, lw1_ref, lb1_ref,
             lw2_ref, lb2_ref, lw3_ref, lb3_ref, act_ref, part_ref):
        aggf = a_ref[...].reshape(bb * n, out_ch)
        posb = jnp.broadcast_to(pos_ref[...][None], (bb, n, p))
        s3 = jnp.concatenate([s_ref[...], posb], axis=-1)
        sf = s3.reshape(bb * n, in_ch)
        root = jnp.dot(sf, rw_ref[...], preferred_element_type=jnp.float32)
        out1 = jnp.maximum(root + rb_ref[...] + aggf, 0.0)

        t = s3[:, :, 1:2]
        total = jnp.sum(t, axis=1, keepdims=True)
        totf = jnp.broadcast_to(total, (bb, n, 1)).reshape(bb * n, 1)

        lw1v = lw1_ref[...]
        x1 = (jnp.dot(out1, lw1v[:out_ch], preferred_element_type=jnp.float32)
              + totf * lw1v[out_ch:out_ch + 1]
              + jnp.dot(sf, lw1v[out_ch + 1:], preferred_element_type=jnp.float32)
              + lb1_ref[...])
        x1 = jnp.where(x1 > 0, x1, 0.01 * x1)
        x2 = jnp.dot(x1, lw2_ref[...], preferred_element_type=jnp.float32) + lb2_ref[...]
        x2 = jnp.where(x2 > 0, x2, 0.01 * x2)
        x3 = jnp.dot(x2, lw3_ref[...], preferred_element_type=jnp.float32) + lb3_ref[...]
        conc = jnp.maximum(x3, 0.0) + jnp.log(1.0 + jnp.exp(-jnp.abs(x3)))
        conc3 = conc.reshape(bb, n, 1)
        denom = jnp.sum(conc3, axis=1, keepdims=True) + 1e-20
        act_ref[...] = conc3 / denom
        part_ref[...] = jnp.full((1, 1, 128), jnp.sum(jnp.abs(conc3)),
                                 jnp.float32)

    return pl.pallas_call(
        body,
        grid=(nb,),
        in_specs=[
            pl.BlockSpec((bb, n, c_raw), lambda i: (i, 0, 0)),
            pl.BlockSpec((n, p), lambda i: (0, 0)),
            pl.BlockSpec((bb, n, out_ch), lambda i: (i, 0, 0)),
            pl.BlockSpec((in_ch, out_ch), lambda i: (0, 0)),
            pl.BlockSpec((1, out_ch), lambda i: (0, 0)),
            pl.BlockSpec((in_ch + out_ch + 1, hid), lambda i: (0, 0)),
            pl.BlockSpec((1, hid), lambda i: (0, 0)),
            pl.BlockSpec((hid, hid), lambda i: (0, 0)),
            pl.BlockSpec((1, hid), lambda i: (0, 0)),
            pl.BlockSpec((hid, 1), lambda i: (0, 0)),
            pl.BlockSpec((1, 1), lambda i: (0, 0)),
        ],
        out_specs=[
            pl.BlockSpec((bb, n, 1), lambda i: (i, 0, 0)),
            pl.BlockSpec((1, 1, 128), lambda i: (i, 0, 0)),
        ],
        out_shape=[
            jax.ShapeDtypeStruct((b, n, 1), jnp.float32),
            jax.ShapeDtypeStruct((nb, 1, 128), jnp.float32),
        ],
    )(state, pos_feat, agg3, root_w, root_b, lw1, lb1, lw2, lb2, lw3, lb3)


def kernel(state, edge_index, edge_attr, pos_feat, w1, b1, w2, b2,
           root_w, root_b, lw1, lb1, lw2, lb2, lw3, lb3):
    b, n, c_raw = state.shape
    p = pos_feat.shape[-1]
    in_ch = c_raw + p
    out_ch = root_w.shape[1]
    e = edge_index.shape[1]
    hid = lw2.shape[0]
    npair = n * n
    nw = _NC * _NS

    epad = -(-e // _LANES) * _LANES
    rpt = -(-npair // (nw * 8)) * 8
    ppad = nw * rpt

    src = edge_index[0].astype(jnp.int32)
    dst = edge_index[1].astype(jnp.int32)
    pid = src * n + dst
    pid_pad = jnp.concatenate(
        [pid, jnp.full((epad - e,), npair, jnp.int32)])
    attr_pad = jnp.concatenate(
        [edge_attr[:, 0], jnp.zeros((epad - e,), jnp.float32)])

    hraw = _sc_pair_accum(attr_pad, pid_pad, w1.reshape(_LANES),
                          b1.reshape(_LANES), ppad=ppad, epad=epad, rpt=rpt)

    w2r = w2.reshape(_LANES, in_ch, out_ch).transpose(1, 0, 2)
    b2r = b2.reshape(in_ch, 1, out_ch)
    wflat = _tc_build_w(hraw, w2r, b2r, in_ch=in_ch, out_ch=out_ch,
                        ppad=ppad, npair=npair)
    wbig = wflat.reshape(in_ch * n, n * out_ch)

    pos_b = jnp.broadcast_to(pos_feat[None], (b, n, p))
    s = jnp.concatenate([state, pos_b], axis=-1)
    st2 = s.transpose(0, 2, 1).reshape(b, in_ch * n).astype(jnp.bfloat16)
    agg = _tc_matmul(st2, wbig, bb=64)
    agg3 = agg.reshape(b, n, out_ch)

    act3, parts = _tc_epilogue(
        state, pos_feat, agg3, root_w, root_b.reshape(1, out_ch), lw1,
        lb1.reshape(1, hid), lw2, lb2.reshape(1, hid), lw3,
        lb3.reshape(1, 1), bb=64)
    action = act3.reshape(b, n)
    regularize = jnp.sum(parts[:, 0, 0]) / (b * n)
    return (action, regularize)


# speedup vs baseline: 1.0730x; 1.0006x over previous
import functools

import jax
import jax.numpy as jnp
from jax import lax
from jax.experimental import pallas as pl
from jax.experimental.pallas import tpu as pltpu
from jax.experimental.pallas import tpu_sc as plsc

_NC = 2
_NS = 16
_LANES = 16


def _sc_pair_accum(attr_pad, pid_pad, w1v, b1v, *, ppad, epad, rpt):
    mesh = plsc.VectorSubcoreMesh(core_axis_name="c", subcore_axis_name="s")

    @functools.partial(
        pl.kernel,
        mesh=mesh,
        out_type=jax.ShapeDtypeStruct((ppad, 2 * _LANES), jnp.float32),
        scratch_types=[
            pltpu.VMEM((epad,), jnp.int32),
            pltpu.VMEM((epad,), jnp.float32),
            pltpu.VMEM((rpt + 8, 2 * _LANES), jnp.float32),
            pltpu.VMEM((_LANES,), jnp.float32),
            pltpu.VMEM((_LANES,), jnp.float32),
        ],
    )
    def k(attr_hbm, pid_hbm, w1_hbm, b1_hbm, out_hbm,
          pid_v, attr_v, acc_v, w1_v, b1_v):
        c = lax.axis_index("c")
        s = lax.axis_index("s")
        wid = c * _NS + s
        rbase = wid * rpt
        pltpu.sync_copy(attr_hbm, attr_v)
        pltpu.sync_copy(pid_hbm, pid_v)
        pltpu.sync_copy(w1_hbm, w1_v)
        pltpu.sync_copy(b1_hbm, b1_v)
        w1r = w1_v[...]
        b1r = b1_v[...]
        cntv = jnp.where(lax.iota(jnp.int32, _LANES) == 0, 1.0, 0.0)
        zv = jnp.zeros((_LANES,), jnp.float32)

        def zbody(r, carry):
            acc_v[r, 0:_LANES] = zv
            acc_v[r, _LANES:2 * _LANES] = zv
            return carry
        lax.fori_loop(0, rpt + 8, zbody, 0)

        def ebody(g, carry):
            av = attr_v[pl.ds(g * _LANES, _LANES)]
            pv = pid_v[pl.ds(g * _LANES, _LANES)]
            lv = pv - rbase
            inr = (lv >= 0) & (lv < rpt)
            iv = jnp.where(inr, lv, rpt)
            for j in range(_LANES):
                a = jnp.full((_LANES,), av[j], jnp.float32)
                h = jnp.maximum(a * w1r + b1r, 0.0)
                idx = iv[j]
                plsc.addupdate(acc_v.at[idx, 0:_LANES], h)
                plsc.addupdate(acc_v.at[idx, _LANES:2 * _LANES], cntv)
            return carry
        lax.fori_loop(0, epad // _LANES, ebody, 0)

        pltpu.sync_copy(acc_v.at[pl.ds(0, rpt)],
                        out_hbm.at[pl.ds(rbase, rpt)])

    return k(attr_pad, pid_pad, w1v, b1v)


def _tc_build_w(hraw, w2r, b2r, *, in_ch, out_ch, ppad, npair):
    ic = 4

    def body(h_ref, w2_ref, b2_ref, o_ref):
        h = h_ref[...]
        h16 = h[:, :_LANES]
        cnt = h[:, _LANES:_LANES + 1]
        for j in range(ic):
            wi = jnp.dot(h16, w2_ref[j], preferred_element_type=jnp.float32)
            o_ref[j] = (wi + cnt * b2_ref[j])[:o_ref.shape[1]].astype(
                jnp.bfloat16)

    return pl.pallas_call(
        body,
        grid=(in_ch // ic,),
        in_specs=[
            pl.BlockSpec((ppad, 2 * _LANES), lambda i: (0, 0)),
            pl.BlockSpec((ic, _LANES, out_ch), lambda i: (i, 0, 0)),
            pl.BlockSpec((ic, 1, out_ch), lambda i: (i, 0, 0)),
        ],
        out_specs=pl.BlockSpec((ic, npair, out_ch), lambda i: (i, 0, 0)),
        out_shape=jax.ShapeDtypeStruct((in_ch, npair, out_ch), jnp.bfloat16),
    )(hraw, w2r, b2r)


def _tc_matmul(st2, wbig, *, bb):
    b, k = st2.shape
    _, nxo = wbig.shape

    def body(x_ref, w_ref, o_ref):
        o_ref[...] = jnp.dot(x_ref[...], w_ref[...],
                             preferred_element_type=jnp.float32)

    return pl.pallas_call(
        body,
        grid=(b // bb,),
        in_specs=[
            pl.BlockSpec((bb, k), lambda i: (i, 0)),
            pl.BlockSpec((k, nxo), lambda i: (0, 0)),
        ],
        out_specs=pl.BlockSpec((bb, nxo), lambda i: (i, 0)),
        out_shape=jax.ShapeDtypeStruct((b, nxo), jnp.float32),
    )(st2, wbig)


def _tc_epilogue(state, pos_feat, agg3, root_w, root_b, lw1, lb1, lw2, lb2,
                 lw3, lb3, *, bb):
    b, n, c_raw = state.shape
    p = pos_feat.shape[-1]
    in_ch = c_raw + p
    out_ch = agg3.shape[-1]
    hid = lw2.shape[0]
    nb = b // bb

    def body(s_ref, pos_ref, a_ref, rw_ref, rb_ref, lw1_ref, lb1_ref,
             lw2_ref, lb2_ref, lw3_ref, lb3_ref, act_ref, part_ref):
        aggf = a_ref[...].reshape(bb * n, out_ch)
        posb = jnp.broadcast_to(pos_ref[...][None], (bb, n, p))
        s3 = jnp.concatenate([s_ref[...], posb], axis=-1)
        sf = s3.reshape(bb * n, in_ch)
        root = jnp.dot(sf, rw_ref[...], preferred_element_type=jnp.float32)
        out1 = jnp.maximum(root + rb_ref[...] + aggf, 0.0)

        t = s3[:, :, 1:2]
        total = jnp.sum(t, axis=1, keepdims=True)
        totf = jnp.broadcast_to(total, (bb, n, 1)).reshape(bb * n, 1)

        lw1v = lw1_ref[...]
        x1 = (jnp.dot(out1, lw1v[:out_ch], preferred_element_type=jnp.float32)
              + totf * lw1v[out_ch:out_ch + 1]
              + jnp.dot(sf, lw1v[out_ch + 1:], preferred_element_type=jnp.float32)
              + lb1_ref[...])
        x1 = jnp.where(x1 > 0, x1, 0.01 * x1)
        x2 = jnp.dot(x1, lw2_ref[...], preferred_element_type=jnp.float32) + lb2_ref[...]
        x2 = jnp.where(x2 > 0, x2, 0.01 * x2)
        x3 = jnp.dot(x2, lw3_ref[...], preferred_element_type=jnp.float32) + lb3_ref[...]
        conc = jnp.maximum(x3, 0.0) + jnp.log(1.0 + jnp.exp(-jnp.abs(x3)))
        conc3 = conc.reshape(bb, n, 1)
        denom = jnp.sum(conc3, axis=1, keepdims=True) + 1e-20
        act_ref[...] = conc3 / denom
        part_ref[...] = jnp.full((1, 1, 128), jnp.sum(jnp.abs(conc3)),
                                 jnp.float32)

    return pl.pallas_call(
        body,
        grid=(nb,),
        in_specs=[
            pl.BlockSpec((bb, n, c_raw), lambda i: (i, 0, 0)),
            pl.BlockSpec((n, p), lambda i: (0, 0)),
            pl.BlockSpec((bb, n, out_ch), lambda i: (i, 0, 0)),
            pl.BlockSpec((in_ch, out_ch), lambda i: (0, 0)),
            pl.BlockSpec((1, out_ch), lambda i: (0, 0)),
            pl.BlockSpec((in_ch + out_ch + 1, hid), lambda i: (0, 0)),
            pl.BlockSpec((1, hid), lambda i: (0, 0)),
            pl.BlockSpec((hid, hid), lambda i: (0, 0)),
            pl.BlockSpec((1, hid), lambda i: (0, 0)),
            pl.BlockSpec((hid, 1), lambda i: (0, 0)),
            pl.BlockSpec((1, 1), lambda i: (0, 0)),
        ],
        out_specs=[
            pl.BlockSpec((bb, n, 1), lambda i: (i, 0, 0)),
            pl.BlockSpec((1, 1, 128), lambda i: (i, 0, 0)),
        ],
        out_shape=[
            jax.ShapeDtypeStruct((b, n, 1), jnp.float32),
            jax.ShapeDtypeStruct((nb, 1, 128), jnp.float32),
        ],
    )(state, pos_feat, agg3, root_w, root_b, lw1, lb1, lw2, lb2, lw3, lb3)


def kernel(state, edge_index, edge_attr, pos_feat, w1, b1, w2, b2,
           root_w, root_b, lw1, lb1, lw2, lb2, lw3, lb3):
    b, n, c_raw = state.shape
    p = pos_feat.shape[-1]
    in_ch = c_raw + p
    out_ch = root_w.shape[1]
    e = edge_index.shape[1]
    hid = lw2.shape[0]
    npair = n * n
    nw = _NC * _NS

    epad = -(-e // _LANES) * _LANES
    rpt = -(-npair // (nw * 8)) * 8
    ppad = nw * rpt

    src = edge_index[0].astype(jnp.int32)
    dst = edge_index[1].astype(jnp.int32)
    pid = src * n + dst
    pid_pad = jnp.concatenate(
        [pid, jnp.full((epad - e,), npair, jnp.int32)])
    attr_pad = jnp.concatenate(
        [edge_attr[:, 0], jnp.zeros((epad - e,), jnp.float32)])

    hraw = _sc_pair_accum(attr_pad, pid_pad, w1.reshape(_LANES),
                          b1.reshape(_LANES), ppad=ppad, epad=epad, rpt=rpt)

    w2r = w2.reshape(_LANES, in_ch, out_ch).transpose(1, 0, 2)
    b2r = b2.reshape(in_ch, 1, out_ch)
    wflat = _tc_build_w(hraw, w2r, b2r, in_ch=in_ch, out_ch=out_ch,
                        ppad=ppad, npair=npair)
    wbig = wflat.reshape(in_ch * n, n * out_ch)

    pos_b = jnp.broadcast_to(pos_feat[None], (b, n, p))
    s = jnp.concatenate([state, pos_b], axis=-1)
    st2 = s.transpose(0, 2, 1).reshape(b, in_ch * n).astype(jnp.bfloat16)
    agg = _tc_matmul(st2, wbig, bb=64)
    agg3 = agg.reshape(b, n, out_ch)

    act3, parts = _tc_epilogue(
        state, pos_feat, agg3, root_w, root_b.reshape(1, out_ch), lw1,
        lb1.reshape(1, hid), lw2, lb2.reshape(1, hid), lw3,
        lb3.reshape(1, 1), bb=32)
    action = act3.reshape(b, n)
    regularize = jnp.sum(parts[:, 0, 0]) / (b * n)
    return (action, regularize)
